# baseline (speedup 1.0000x reference)
import jax
import jax.numpy as jnp
from jax import lax
from jax.experimental import pallas as pl
from jax.experimental.pallas import tpu as pltpu
from jax.experimental.pallas import tpu_sc as plsc

N = 10000
D = 128
E = 320000
DP = D // 2

NC = 2
NS = 16
NW = NC * NS

CHUNK = 80
NCHUNKS = 125
EPW = CHUNK * NCHUNKS
NB = 4
LAST = NCHUNKS - 1

_LANE_DNUMS = lax.GatherDimensionNumbers(
    offset_dims=(), collapsed_slice_dims=(0,), start_index_map=(0,))


def _lane_broadcast(vec, j):
    idx = jnp.full((16, 1), j, dtype=jnp.int32)
    return lax.gather(vec, idx, _LANE_DNUMS, (1,),
                      mode=lax.GatherScatterMode.PROMISE_IN_BOUNDS)


def _pack_rows(x):
    xb = x.astype(jnp.bfloat16).reshape(-1, 4, 2, 16)
    xb = jnp.swapaxes(xb, 2, 3)
    return lax.bitcast_convert_type(xb, jnp.int32).reshape(-1, DP)


def _agg_body(x_hbm, src_hbm, dst_hbm, w_hbm, out_hbm,
              sb0, sb1, sb2, sb3, wb0, wb1, wb2, wb3,
              db0, db1, db2, db3, g0, g1, g2, g3, s0, s1, acc_sh,
              gs0, gs1, gs2, gs3, ss0, ss1,
              es0, es1, es2, es3, ds0, ds1, ds2, ds3):
    c = lax.axis_index("c")
    s = lax.axis_index("s")
    wid = s * NC + c
    ebase = wid * EPW
    SB = (sb0, sb1, sb2, sb3)
    WB = (wb0, wb1, wb2, wb3)
    DB = (db0, db1, db2, db3)
    G = (g0, g1, g2, g3)
    S = (s0, s1)
    GS = (gs0, gs1, gs2, gs3)
    SS = (ss0, ss1)
    ES = (es0, es1, es2, es3)
    DS = (ds0, ds1, ds2, ds3)

    def sw_start(k, b):
        pltpu.async_copy(src_hbm.at[pl.ds(ebase + k * CHUNK, CHUNK)],
                         SB[b], ES[b])
        pltpu.async_copy(w_hbm.at[pl.ds(ebase + k * CHUNK, CHUNK)],
                         WB[b], ES[b])

    def sw_wait(k, b):
        pltpu.make_async_copy(src_hbm.at[pl.ds(ebase, CHUNK)],
                              SB[b], ES[b]).wait()
        pltpu.make_async_copy(w_hbm.at[pl.ds(ebase, CHUNK)],
                              WB[b], ES[b]).wait()

    def d_start(k, b):
        pltpu.async_copy(dst_hbm.at[pl.ds(ebase + k * CHUNK, CHUNK)],
                         DB[b], DS[b])

    def d_wait(k, b):
        pltpu.make_async_copy(dst_hbm.at[pl.ds(ebase, CHUNK)],
                              DB[b], DS[b]).wait()

    def g_start(k, b):
        pltpu.async_copy(x_hbm.at[SB[b]], G[b], GS[b])

    def g_wait(k, b):
        pltpu.make_async_copy(x_hbm.at[SB[b]], G[b], GS[b]).wait()

    def s_start(p, db):
        pltpu.async_copy(S[p], acc_sh.at[DB[db]], SS[p], add=True)

    def s_wait(p, db):
        pltpu.make_async_copy(S[p], acc_sh.at[DB[db]], SS[p]).wait()

    def mul(k, b, p):
        def mul_group(g, c2):
            wv = WB[b][pl.ds(g * 16, 16)]
            for j in range(16):
                wb = _lane_broadcast(wv, j)
                e = g * 16 + j
                for q in range(4):
                    v = G[b][e, pl.ds(q * 16, 16)]
                    lo = lax.bitcast_convert_type(v << 16, jnp.float32)
                    hi = lax.bitcast_convert_type(v & jnp.int32(-65536),
                                                  jnp.float32)
                    S[p][e, pl.ds(q * 32, 16)] = lo * wb
                    S[p][e, pl.ds(q * 32 + 16, 16)] = hi * wb
            return c2
        lax.fori_loop(0, CHUNK // 16, mul_group, 0)

    zv = jnp.zeros((16,), jnp.float32)

    def zfill(z, c2):
        for d in range(8):
            s0[z, pl.ds(d * 16, 16)] = zv
        return c2
    lax.fori_loop(0, CHUNK, zfill, 0)
    rows = s * 624
    for q in range(7):
        pltpu.sync_copy(s0, acc_sh.at[pl.ds(rows + q * CHUNK, CHUNK)])

    @pl.when(s < 15)
    def _():
        pltpu.sync_copy(s0.at[pl.ds(0, 64)],
                        acc_sh.at[pl.ds(rows + 7 * CHUNK, 64)])

    @pl.when(s == 15)
    def _():
        pltpu.sync_copy(s0, acc_sh.at[pl.ds(rows + 7 * CHUNK, CHUNK)])

    plsc.subcore_barrier()

    sw_start(0, 0)
    sw_start(1, 1)
    sw_start(2, 2)
    sw_start(3, 3)
    d_start(0, 0)
    d_start(1, 1)
    sw_wait(0, 0)
    g_start(0, 0)
    sw_wait(1, 1)
    g_start(1, 1)
    sw_wait(2, 2)
    g_start(2, 2)

    def step(k, b):
        p = b % 2
        b3 = (b + 3) % NB
        b2 = (b + 2) % NB
        g_wait(k, b)

        @pl.when(k >= 2)
        def _(k=k, p=p, b2=b2):
            s_wait(p, b2)

        @pl.when(k < NCHUNKS - 2)
        def _(k=k, b2=b2):
            d_start(k + 2, b2)

        @pl.when(k < NCHUNKS - 3)
        def _(k=k, b3=b3):
            sw_wait(k + 3, b3)
            g_start(k + 3, b3)

        mul(k, b, p)

        @pl.when(k < NCHUNKS - NB)
        def _(k=k, b=b):
            sw_start(k + NB, b)

        d_wait(k, b)
        s_start(p, b)

    def outer(i, carry):
        k0 = i * NB
        for b in range(NB):
            step(k0 + b, b)
        return carry

    lax.fori_loop(0, LAST // NB, outer, 0)
    step(LAST, LAST % NB)
    s_wait((NCHUNKS - 2) % 2, (NCHUNKS - 2) % NB)
    s_wait((NCHUNKS - 1) % 2, (NCHUNKS - 1) % NB)

    plsc.subcore_barrier()

    @pl.when(s < 15)
    def _():
        pltpu.sync_copy(acc_sh.at[pl.ds(s * 624, 624)],
                        out_hbm.at[c, pl.ds(s * 624, 624)])

    @pl.when(s == 15)
    def _():
        pltpu.sync_copy(acc_sh.at[pl.ds(15 * 624, 640)],
                        out_hbm.at[c, pl.ds(15 * 624, 640)])


_agg_call = pl.kernel(
    _agg_body,
    out_type=jax.ShapeDtypeStruct((NC, N, D), jnp.float32),
    mesh=plsc.VectorSubcoreMesh(core_axis_name="c", subcore_axis_name="s"),
    compiler_params=pltpu.CompilerParams(use_tc_tiling_on_sc=False),
    scratch_types=(
        [pltpu.VMEM((CHUNK,), jnp.int32) for _ in range(NB)]
        + [pltpu.VMEM((CHUNK,), jnp.float32) for _ in range(NB)]
        + [pltpu.VMEM((CHUNK,), jnp.int32) for _ in range(NB)]
        + [pltpu.VMEM((CHUNK, DP), jnp.int32) for _ in range(NB)]
        + [pltpu.VMEM((CHUNK, D), jnp.float32) for _ in range(2)]
        + [pltpu.VMEM_SHARED((N, D), jnp.float32)]
        + [pltpu.SemaphoreType.DMA for _ in range(NB + 2 + 2 * NB)]
    ),
)


def _dense(p, W, b, relu):
    def body(p_ref, w_ref, b_ref, o_ref):
        acc = p_ref[0] + p_ref[1]
        r = jnp.dot(acc, w_ref[...], preferred_element_type=jnp.float32,
                    precision=lax.Precision.HIGHEST) + b_ref[...]
        o_ref[...] = jnp.maximum(r, 0.0) if relu else r

    R = 2000
    return pl.pallas_call(
        body,
        grid=(N // R,),
        in_specs=[
            pl.BlockSpec((2, R, D), lambda i: (0, i, 0)),
            pl.BlockSpec((D, D), lambda i: (0, 0)),
            pl.BlockSpec((1, D), lambda i: (0, 0)),
        ],
        out_specs=pl.BlockSpec((R, D), lambda i: (i, 0)),
        out_shape=jax.ShapeDtypeStruct((N, D), jnp.float32),
    )(p, W, b.reshape(1, D))


def kernel(x, edge_index, edge_weight, W1, b1, W2, b2):
    src = edge_index[0].astype(jnp.int32)
    dst = edge_index[1].astype(jnp.int32)
    w = edge_weight.astype(jnp.float32)

    p1 = _agg_call(_pack_rows(x), src, dst, w)
    h = _dense(p1, W1, b1, relu=True)
    p2 = _agg_call(_pack_rows(h), src, dst, w)
    return _dense(p2, W2, b2, relu=False)


# speedup vs baseline: 2.4300x; 2.4300x over previous
import jax
import jax.numpy as jnp
from jax import lax
from jax.experimental import pallas as pl
from jax.experimental.pallas import tpu as pltpu
from jax.experimental.pallas import tpu_sc as plsc

N = 10000
D = 128
E = 320000

NC = 2
NS = 16
NW = NC * NS

CHUNK = 80
NCHUNKS = 125
EPW = CHUNK * NCHUNKS
NB = 4
LAST = NCHUNKS - 1

_LANE_DNUMS = lax.GatherDimensionNumbers(
    offset_dims=(), collapsed_slice_dims=(0,), start_index_map=(0,))


def _lane_broadcast(vec, j):
    idx = jnp.full((16, 1), j, dtype=jnp.int32)
    return lax.gather(vec, idx, _LANE_DNUMS, (1,),
                      mode=lax.GatherScatterMode.PROMISE_IN_BOUNDS)


def _agg_body(x_hbm, src_hbm, dst_hbm, w_hbm, out_hbm,
              sb0, sb1, sb2, sb3, wb0, wb1, wb2, wb3,
              db0, db1, db2, db3, r0, r1, r2, r3, acc_sh,
              gs0, gs1, gs2, gs3, ss0, ss1, ss2, ss3,
              es0, es1, es2, es3, ds0, ds1, ds2, ds3):
    c = lax.axis_index("c")
    s = lax.axis_index("s")
    wid = s * NC + c
    ebase = wid * EPW
    SB = (sb0, sb1, sb2, sb3)
    WB = (wb0, wb1, wb2, wb3)
    DB = (db0, db1, db2, db3)
    R = (r0, r1, r2, r3)
    GS = (gs0, gs1, gs2, gs3)
    SS = (ss0, ss1, ss2, ss3)
    ES = (es0, es1, es2, es3)
    DS = (ds0, ds1, ds2, ds3)

    def sw_start(k, b):
        pltpu.async_copy(src_hbm.at[pl.ds(ebase + k * CHUNK, CHUNK)],
                         SB[b], ES[b])
        pltpu.async_copy(w_hbm.at[pl.ds(ebase + k * CHUNK, CHUNK)],
                         WB[b], ES[b])

    def sw_wait(k, b):
        pltpu.make_async_copy(src_hbm.at[pl.ds(ebase, CHUNK)],
                              SB[b], ES[b]).wait()
        pltpu.make_async_copy(w_hbm.at[pl.ds(ebase, CHUNK)],
                              WB[b], ES[b]).wait()

    def d_start(k, b):
        pltpu.async_copy(dst_hbm.at[pl.ds(ebase + k * CHUNK, CHUNK)],
                         DB[b], DS[b])

    def d_wait(k, b):
        pltpu.make_async_copy(dst_hbm.at[pl.ds(ebase, CHUNK)],
                              DB[b], DS[b]).wait()

    def g_start(k, b):
        pltpu.async_copy(x_hbm.at[SB[b]], R[b], GS[b])

    def g_wait(k, b):
        pltpu.make_async_copy(x_hbm.at[SB[b]], R[b], GS[b]).wait()

    def s_start(k, b):
        pltpu.async_copy(R[b], acc_sh.at[DB[b]], SS[b], add=True)

    def s_wait(k, b):
        pltpu.make_async_copy(R[b], acc_sh.at[DB[b]], SS[b]).wait()

    def mul(k, b):
        def mul_group(g, c2):
            wv = WB[b][pl.ds(g * 16, 16)]
            for j in range(16):
                wb = _lane_broadcast(wv, j)
                e = g * 16 + j
                for d in range(8):
                    sl = pl.ds(d * 16, 16)
                    R[b][e, sl] = R[b][e, sl] * wb
            return c2
        lax.fori_loop(0, CHUNK // 16, mul_group, 0)

    zv = jnp.zeros((16,), jnp.float32)

    def zfill(z, c2):
        for d in range(8):
            r3[z, pl.ds(d * 16, 16)] = zv
        return c2
    lax.fori_loop(0, CHUNK, zfill, 0)
    rows = s * 624
    for q in range(7):
        pltpu.sync_copy(r3, acc_sh.at[pl.ds(rows + q * CHUNK, CHUNK)])

    @pl.when(s < 15)
    def _():
        pltpu.sync_copy(r3.at[pl.ds(0, 64)],
                        acc_sh.at[pl.ds(rows + 7 * CHUNK, 64)])

    @pl.when(s == 15)
    def _():
        pltpu.sync_copy(r3, acc_sh.at[pl.ds(rows + 7 * CHUNK, CHUNK)])

    plsc.subcore_barrier()

    sw_start(0, 0)
    sw_start(1, 1)
    sw_start(2, 2)
    sw_start(3, 3)
    d_start(0, 0)
    d_start(1, 1)
    sw_wait(0, 0)
    g_start(0, 0)
    sw_wait(1, 1)
    g_start(1, 1)

    def step(k, b):
        b2 = (b + 2) % NB
        g_wait(k, b)

        @pl.when(k >= 2)
        def _(k=k, b2=b2):
            s_wait(k - 2, b2)

        @pl.when(k < NCHUNKS - 2)
        def _(k=k, b2=b2):
            d_start(k + 2, b2)
            sw_wait(k + 2, b2)
            g_start(k + 2, b2)

        mul(k, b)

        @pl.when(k < NCHUNKS - NB)
        def _(k=k, b=b):
            sw_start(k + NB, b)

        d_wait(k, b)
        s_start(k, b)

    def outer(i, carry):
        k0 = i * NB
        for b in range(NB):
            step(k0 + b, b)
        return carry

    lax.fori_loop(0, LAST // NB, outer, 0)
    step(LAST, LAST % NB)
    s_wait(NCHUNKS - 2, (NCHUNKS - 2) % NB)
    s_wait(NCHUNKS - 1, (NCHUNKS - 1) % NB)

    plsc.subcore_barrier()

    @pl.when(s < 15)
    def _():
        pltpu.sync_copy(acc_sh.at[pl.ds(s * 624, 624)],
                        out_hbm.at[c, pl.ds(s * 624, 624)])

    @pl.when(s == 15)
    def _():
        pltpu.sync_copy(acc_sh.at[pl.ds(15 * 624, 640)],
                        out_hbm.at[c, pl.ds(15 * 624, 640)])


_agg_call = pl.kernel(
    _agg_body,
    out_type=jax.ShapeDtypeStruct((NC, N, D), jnp.float32),
    mesh=plsc.VectorSubcoreMesh(core_axis_name="c", subcore_axis_name="s"),
    scratch_types=(
        [pltpu.VMEM((CHUNK,), jnp.int32) for _ in range(NB)]
        + [pltpu.VMEM((CHUNK,), jnp.float32) for _ in range(NB)]
        + [pltpu.VMEM((CHUNK,), jnp.int32) for _ in range(NB)]
        + [pltpu.VMEM((CHUNK, D), jnp.float32) for _ in range(NB)]
        + [pltpu.VMEM_SHARED((N, D), jnp.float32)]
        + [pltpu.SemaphoreType.DMA for _ in range(4 * NB)]
    ),
)


def _dense(p, W, b, relu):
    def body(p_ref, w_ref, b_ref, o_ref):
        acc = p_ref[0] + p_ref[1]
        r = jnp.dot(acc, w_ref[...], preferred_element_type=jnp.float32,
                    precision=lax.Precision.HIGHEST) + b_ref[...]
        o_ref[...] = jnp.maximum(r, 0.0) if relu else r

    R = 2000
    return pl.pallas_call(
        body,
        grid=(N // R,),
        in_specs=[
            pl.BlockSpec((2, R, D), lambda i: (0, i, 0)),
            pl.BlockSpec((D, D), lambda i: (0, 0)),
            pl.BlockSpec((1, D), lambda i: (0, 0)),
        ],
        out_specs=pl.BlockSpec((R, D), lambda i: (i, 0)),
        out_shape=jax.ShapeDtypeStruct((N, D), jnp.float32),
    )(p, W, b.reshape(1, D))


def kernel(x, edge_index, edge_weight, W1, b1, W2, b2):
    src = edge_index[0].astype(jnp.int32)
    dst = edge_index[1].astype(jnp.int32)
    w = edge_weight.astype(jnp.float32)

    p1 = _agg_call(x, src, dst, w)
    h = _dense(p1, W1, b1, relu=True)
    p2 = _agg_call(h, src, dst, w)
    return _dense(p2, W2, b2, relu=False)


# speedup vs baseline: 2.4329x; 1.0012x over previous
import jax
import jax.numpy as jnp
from jax import lax
from jax.experimental import pallas as pl
from jax.experimental.pallas import tpu as pltpu
from jax.experimental.pallas import tpu_sc as plsc

N = 10000
D = 128
E = 320000

NC = 2
NS = 16
NW = NC * NS

CHUNK = 80
NCHUNKS = 125
EPW = CHUNK * NCHUNKS
NB = 4
GW = CHUNK * NB

_LANE_DNUMS = lax.GatherDimensionNumbers(
    offset_dims=(), collapsed_slice_dims=(0,), start_index_map=(0,))


def _lane_broadcast(vec, j):
    idx = jnp.full((16, 1), j, dtype=jnp.int32)
    return lax.gather(vec, idx, _LANE_DNUMS, (1,),
                      mode=lax.GatherScatterMode.PROMISE_IN_BOUNDS)


def _agg_body(x_hbm, src_hbm, dst_hbm, w_hbm, out_hbm,
              sbb0, sbb1, wbb0, wbb1,
              db0, db1, db2, db3, r0, r1, r2, r3, acc_sh,
              gs0, gs1, gs2, gs3, ss0, ss1, ss2, ss3,
              eb0, eb1, ds0, ds1, ds2, ds3):
    c = lax.axis_index("c")
    s = lax.axis_index("s")
    wid = s * NC + c
    ebase = wid * EPW
    SBB = (sbb0, sbb1)
    WBB = (wbb0, wbb1)
    DB = (db0, db1, db2, db3)
    R = (r0, r1, r2, r3)
    GS = (gs0, gs1, gs2, gs3)
    SS = (ss0, ss1, ss2, ss3)
    EB = (eb0, eb1)
    DS = (ds0, ds1, ds2, ds3)

    def swb_start(grp, pp):
        pltpu.async_copy(src_hbm.at[pl.ds(ebase + grp * GW, GW)],
                         SBB[pp], EB[pp])
        pltpu.async_copy(w_hbm.at[pl.ds(ebase + grp * GW, GW)],
                         WBB[pp], EB[pp])

    def swb_wait(pp):
        pltpu.make_async_copy(src_hbm.at[pl.ds(ebase, GW)],
                              SBB[pp], EB[pp]).wait()
        pltpu.make_async_copy(w_hbm.at[pl.ds(ebase, GW)],
                              WBB[pp], EB[pp]).wait()

    def d_start(k, b):
        pltpu.async_copy(dst_hbm.at[pl.ds(ebase + k * CHUNK, CHUNK)],
                         DB[b], DS[b])

    def d_wait(b):
        pltpu.make_async_copy(dst_hbm.at[pl.ds(ebase, CHUNK)],
                              DB[b], DS[b]).wait()

    def g_start(b, pp, q):
        idx = SBB[pp].at[pl.ds(q * CHUNK, CHUNK)]
        pltpu.async_copy(x_hbm.at[idx], R[b], GS[b])

    def g_wait(b, pp, q):
        idx = SBB[pp].at[pl.ds(q * CHUNK, CHUNK)]
        pltpu.make_async_copy(x_hbm.at[idx], R[b], GS[b]).wait()

    def s_start(b):
        pltpu.async_copy(R[b], acc_sh.at[DB[b]], SS[b], add=True)

    def s_wait(b):
        pltpu.make_async_copy(R[b], acc_sh.at[DB[b]], SS[b]).wait()

    def mul(b, pp, q):
        def mul_group(g, c2):
            wv = WBB[pp][pl.ds(q * CHUNK + g * 16, 16)]
            for j in range(16):
                wb = _lane_broadcast(wv, j)
                e = g * 16 + j
                for d in range(8):
                    sl = pl.ds(d * 16, 16)
                    R[b][e, sl] = R[b][e, sl] * wb
            return c2
        lax.fori_loop(0, CHUNK // 16, mul_group, 0)

    swb_start(0, 0)
    swb_start(1, 1)
    d_start(0, 0)
    d_start(1, 1)
    swb_wait(0)
    g_start(0, 0, 0)
    g_start(1, 0, 1)

    zv = jnp.zeros((16,), jnp.float32)

    def zfill(z, c2):
        for d in range(8):
            r3[z, pl.ds(d * 16, 16)] = zv
        return c2
    lax.fori_loop(0, CHUNK, zfill, 0)
    rows = s * 624
    for q in range(7):
        pltpu.sync_copy(r3, acc_sh.at[pl.ds(rows + q * CHUNK, CHUNK)])

    @pl.when(s < 15)
    def _():
        pltpu.sync_copy(r3.at[pl.ds(0, 64)],
                        acc_sh.at[pl.ds(rows + 7 * CHUNK, 64)])

    @pl.when(s == 15)
    def _():
        pltpu.sync_copy(r3, acc_sh.at[pl.ds(rows + 7 * CHUNK, CHUNK)])

    plsc.subcore_barrier()

    def step(k, b8):
        b = b8 % NB
        q = b8 % NB
        pp = (b8 // NB) % 2
        b2 = (b8 + 2) % NB
        q2 = (b8 + 2) % NB
        pp2 = ((b8 + 2) // NB) % 2

        g_wait(b, pp, q)

        @pl.when(k >= 2)
        def _(b2=b2):
            s_wait(b2)

        @pl.when(k < NCHUNKS - 2)
        def _(k=k, b2=b2, pp2=pp2, q2=q2):
            d_start(k + 2, b2)
            if q2 == 0:
                swb_wait(pp2)
            g_start(b2, pp2, q2)

        mul(b, pp, q)

        if b8 % NB == NB - 1:
            @pl.when(k < NCHUNKS - 5)
            def _(k=k, pp=pp):
                swb_start(k // NB + 2, pp)

        d_wait(b)
        s_start(b)

    def outer(i, carry):
        k0 = i * 8
        for b8 in range(8):
            step(k0 + b8, b8)
        return carry

    lax.fori_loop(0, 15, outer, 0)
    for k in range(120, NCHUNKS):
        step(k, k % 8)
    s_wait((NCHUNKS - 2) % NB)
    s_wait((NCHUNKS - 1) % NB)

    plsc.subcore_barrier()

    @pl.when(s < 15)
    def _():
        pltpu.sync_copy(acc_sh.at[pl.ds(s * 624, 624)],
                        out_hbm.at[c, pl.ds(s * 624, 624)])

    @pl.when(s == 15)
    def _():
        pltpu.sync_copy(acc_sh.at[pl.ds(15 * 624, 640)],
                        out_hbm.at[c, pl.ds(15 * 624, 640)])


_agg_call = pl.kernel(
    _agg_body,
    out_type=jax.ShapeDtypeStruct((NC, N, D), jnp.float32),
    mesh=plsc.VectorSubcoreMesh(core_axis_name="c", subcore_axis_name="s"),
    scratch_types=(
        [pltpu.VMEM((GW,), jnp.int32) for _ in range(2)]
        + [pltpu.VMEM((GW,), jnp.float32) for _ in range(2)]
        + [pltpu.VMEM((CHUNK,), jnp.int32) for _ in range(NB)]
        + [pltpu.VMEM((CHUNK, D), jnp.float32) for _ in range(NB)]
        + [pltpu.VMEM_SHARED((N, D), jnp.float32)]
        + [pltpu.SemaphoreType.DMA for _ in range(3 * NB + 2)]
    ),
)


def _dense(p, W, b, relu):
    def body(p_ref, w_ref, b_ref, o_ref):
        acc = p_ref[0] + p_ref[1]
        r = jnp.dot(acc, w_ref[...], preferred_element_type=jnp.float32,
                    precision=lax.Precision.HIGHEST) + b_ref[...]
        o_ref[...] = jnp.maximum(r, 0.0) if relu else r

    R = 2000
    return pl.pallas_call(
        body,
        grid=(N // R,),
        in_specs=[
            pl.BlockSpec((2, R, D), lambda i: (0, i, 0)),
            pl.BlockSpec((D, D), lambda i: (0, 0)),
            pl.BlockSpec((1, D), lambda i: (0, 0)),
        ],
        out_specs=pl.BlockSpec((R, D), lambda i: (i, 0)),
        out_shape=jax.ShapeDtypeStruct((N, D), jnp.float32),
    )(p, W, b.reshape(1, D))


def kernel(x, edge_index, edge_weight, W1, b1, W2, b2):
    src = edge_index[0].astype(jnp.int32)
    dst = edge_index[1].astype(jnp.int32)
    w = edge_weight.astype(jnp.float32)

    p1 = _agg_call(x, src, dst, w)
    h = _dense(p1, W1, b1, relu=True)
    p2 = _agg_call(h, src, dst, w)
    return _dense(p2, W2, b2, relu=False)
